# baseline (speedup 1.0000x reference)
import jax
import jax.numpy as jnp
from jax import lax
from jax.experimental import pallas as pl
from jax.experimental.pallas import tpu as pltpu
from jax.experimental.pallas import tpu_sc as plsc

_NC, _NS = 2, 16
_NW = _NC * _NS
_TOTAL = 12_800_000
_W = _TOTAL // _NW
_CH = 50_000
_NCHUNK = _W // _CH


def _sc_copy(x_hbm, out_hbm, buf0, buf1, gsem0, gsem1, ssem0, ssem1):
    c = lax.axis_index("c")
    s = lax.axis_index("s")
    wid = s * _NC + c
    base = wid * _W
    bufs = (buf0, buf1)
    gsems = (gsem0, gsem1)
    ssems = (ssem0, ssem1)
    gathers = [None] * _NCHUNK
    scatters = [None] * _NCHUNK
    gathers[0] = pltpu.async_copy(x_hbm.at[pl.ds(base, _CH)], bufs[0], gsems[0])
    for i in range(_NCHUNK):
        b = i % 2
        nb = (i + 1) % 2
        if i + 1 < _NCHUNK:
            if i - 1 >= 0:
                scatters[i - 1].wait()
            gathers[i + 1] = pltpu.async_copy(
                x_hbm.at[pl.ds(base + (i + 1) * _CH, _CH)], bufs[nb], gsems[nb]
            )
        gathers[i].wait()
        scatters[i] = pltpu.async_copy(
            bufs[b], out_hbm.at[pl.ds(base + i * _CH, _CH)], ssems[b]
        )
    scatters[_NCHUNK - 2].wait()
    scatters[_NCHUNK - 1].wait()


def kernel(edge_index):
    E = edge_index.shape[1]
    x = edge_index.reshape(_TOTAL)
    run = pl.kernel(
        _sc_copy,
        out_type=jax.ShapeDtypeStruct((_TOTAL,), jnp.int32),
        mesh=plsc.VectorSubcoreMesh(
            core_axis_name="c",
            subcore_axis_name="s",
            num_cores=_NC,
            num_subcores=_NS,
        ),
        scratch_types=[
            pltpu.VMEM((_CH,), jnp.int32),
            pltpu.VMEM((_CH,), jnp.int32),
            pltpu.SemaphoreType.DMA,
            pltpu.SemaphoreType.DMA,
            pltpu.SemaphoreType.DMA,
            pltpu.SemaphoreType.DMA,
        ],
    )
    out = run(x)
    return out.reshape(2, E)


# speedup vs baseline: 36.2159x; 36.2159x over previous
import jax
import jax.numpy as jnp
from jax.experimental import pallas as pl
from jax.experimental.pallas import tpu as pltpu

_BC = 400_000


def _copy_block(x_ref, o_ref):
    o_ref[...] = x_ref[...]


def kernel(edge_index):
    E = edge_index.shape[1]
    out = pl.pallas_call(
        _copy_block,
        grid=(E // _BC,),
        in_specs=[pl.BlockSpec((2, _BC), lambda i: (0, i))],
        out_specs=pl.BlockSpec((2, _BC), lambda i: (0, i)),
        out_shape=jax.ShapeDtypeStruct((2, E), edge_index.dtype),
    )(edge_index)
    return out


# speedup vs baseline: 37.9134x; 1.0469x over previous
import jax
import jax.numpy as jnp
from jax.experimental import pallas as pl
from jax.experimental.pallas import tpu as pltpu

_BC = 800_000


def _copy_block(x_ref, o_ref):
    o_ref[...] = x_ref[...]


def kernel(edge_index):
    E = edge_index.shape[1]
    out = pl.pallas_call(
        _copy_block,
        grid=(E // _BC,),
        in_specs=[pl.BlockSpec((2, _BC), lambda i: (0, i))],
        out_specs=pl.BlockSpec((2, _BC), lambda i: (0, i)),
        out_shape=jax.ShapeDtypeStruct((2, E), edge_index.dtype),
    )(edge_index)
    return out


# speedup vs baseline: 38.7621x; 1.0224x over previous
import jax
import jax.numpy as jnp
from jax.experimental import pallas as pl
from jax.experimental.pallas import tpu as pltpu

_BC = 1_600_000


def _copy_block(x_ref, o_ref):
    o_ref[...] = x_ref[...]


def kernel(edge_index):
    E = edge_index.shape[1]
    out = pl.pallas_call(
        _copy_block,
        grid=(E // _BC,),
        in_specs=[pl.BlockSpec((2, _BC), lambda i: (0, i))],
        out_specs=pl.BlockSpec((2, _BC), lambda i: (0, i)),
        out_shape=jax.ShapeDtypeStruct((2, E), edge_index.dtype),
    )(edge_index)
    return out
